# baseline (speedup 1.0000x reference)
import functools

import jax
import jax.numpy as jnp
from jax import lax
from jax.experimental import pallas as pl
from jax.experimental.pallas import tpu as pltpu
from jax.experimental.pallas import tpu_sc as plsc

_F = 16
_PACK = 8
_IDX_CHUNK = 128


def _relayout(table_t):
    F, N = table_t.shape
    blk_n = 8192
    assert F == _F and N % _PACK == 0

    def body(in_ref, out_ref):
        x = in_ref[...]
        z = x.reshape(_F, blk_n // _PACK, _PACK).transpose(1, 2, 0)
        out_ref[...] = z.reshape(blk_n // _PACK, _PACK * _F)

    return pl.pallas_call(
        body,
        grid=(pl.cdiv(N, blk_n),),
        in_specs=[pl.BlockSpec((F, blk_n), lambda i: (0, i))],
        out_specs=pl.BlockSpec((blk_n // _PACK, _PACK * _F), lambda i: (i, 0)),
        out_shape=jax.ShapeDtypeStruct((N // _PACK, _PACK * _F), jnp.float32),
    )(table_t)


@functools.cache
def _make_gather(B, R):
    info = plsc.get_sparse_core_info()
    nc, ns = info.num_cores, info.num_subcores
    nw = nc * ns
    b_per_w = B // nw
    n_chunks = b_per_w // _IDX_CHUNK
    mesh = plsc.VectorSubcoreMesh(core_axis_name="c", subcore_axis_name="s")

    @functools.partial(
        pl.kernel,
        mesh=mesh,
        compiler_params=pltpu.CompilerParams(use_tc_tiling_on_sc=False),
        out_type=(
            jax.ShapeDtypeStruct((B, _PACK * _F), jnp.float32),
            jax.ShapeDtypeStruct((B, _PACK * _F), jnp.float32),
        ),
        scratch_types=[
            pltpu.VMEM((n_chunks, _IDX_CHUNK), jnp.int32),
            pltpu.VMEM((n_chunks, _IDX_CHUNK), jnp.int32),
            pltpu.VMEM((b_per_w, _PACK * _F), jnp.float32),
            pltpu.SemaphoreType.DMA,
        ],
    )
    def gather_k(user_hbm, item_hbm, upk_hbm, ipk_hbm, u_out, i_out,
                 idx_v, q_v, rows_v, sem):
        wid = lax.axis_index("s") * nc + lax.axis_index("c")
        base = wid * b_per_w
        for idx_hbm, pk_hbm, out in ((user_hbm, upk_hbm, u_out),
                                     (item_hbm, ipk_hbm, i_out)):
            for j in range(n_chunks):
                pltpu.sync_copy(idx_hbm.at[pl.ds(base + j * _IDX_CHUNK,
                                                 _IDX_CHUNK)], idx_v.at[j])
            for j in range(n_chunks):
                for s in range(_IDX_CHUNK // 16):
                    q_v[j, pl.ds(s * 16, 16)] = (
                        idx_v[j, pl.ds(s * 16, 16)] >> 3)
            copies = [
                pltpu.async_copy(
                    pk_hbm.at[q_v.at[j]],
                    rows_v.at[pl.ds(j * _IDX_CHUNK, _IDX_CHUNK)], sem)
                for j in range(n_chunks)
            ]
            for c in copies:
                c.wait()
            pltpu.sync_copy(rows_v, out.at[pl.ds(base, b_per_w)])

    return gather_k


def _mlp_pallas(gu, gi, u2d, i2d, W1u_big, W1i_big, b1, W2, b2, W3, b3):
    B, L = gu.shape
    blk = 2048
    n1 = W1u_big.shape[1]
    n2 = W2.shape[1]

    def body(gu_ref, gi_ref, u_ref, i_ref, w1u_ref, w1i_ref, b1_ref,
             w2_ref, b2_ref, w3_ref, b3_ref, out_ref):
        su = u_ref[...] & 7
        si = i_ref[...] & 7
        grp = lax.broadcasted_iota(jnp.int32, (blk, L), 1) >> 4
        um = jnp.where(grp == su, gu_ref[...], 0.0)
        im = jnp.where(grp == si, gi_ref[...], 0.0)
        h = (jnp.dot(um, w1u_ref[...], preferred_element_type=jnp.float32)
             + jnp.dot(im, w1i_ref[...], preferred_element_type=jnp.float32)
             + b1_ref[...])
        h = jnp.dot(h, w2_ref[...], preferred_element_type=jnp.float32) \
            + b2_ref[...]
        o = jnp.dot(h, w3_ref[...], preferred_element_type=jnp.float32) \
            + b3_ref[...]
        out_ref[...] = 1.0 / (1.0 + jnp.exp(-o))

    return pl.pallas_call(
        body,
        grid=(B // blk,),
        in_specs=[
            pl.BlockSpec((blk, L), lambda i: (i, 0)),
            pl.BlockSpec((blk, L), lambda i: (i, 0)),
            pl.BlockSpec((blk, 1), lambda i: (i, 0)),
            pl.BlockSpec((blk, 1), lambda i: (i, 0)),
            pl.BlockSpec((L, n1), lambda i: (0, 0)),
            pl.BlockSpec((L, n1), lambda i: (0, 0)),
            pl.BlockSpec((1, n1), lambda i: (0, 0)),
            pl.BlockSpec((n1, n2), lambda i: (0, 0)),
            pl.BlockSpec((1, n2), lambda i: (0, 0)),
            pl.BlockSpec((n2, 1), lambda i: (0, 0)),
            pl.BlockSpec((1, 1), lambda i: (0, 0)),
        ],
        out_specs=pl.BlockSpec((blk, 1), lambda i: (i, 0)),
        out_shape=jax.ShapeDtypeStruct((B, 1), jnp.float32),
    )(gu, gi, u2d, i2d, W1u_big, W1i_big, b1.reshape(1, n1), W2,
      b2.reshape(1, n2), W3, b3.reshape(1, 1))


def kernel(user, item, user_emb, item_emb, W1, b1, W2, b2, W3, b3):
    B = user.shape[0]
    N = user_emb.shape[0]
    upk = _relayout(user_emb.T)
    ipk = _relayout(item_emb.T)
    gather = _make_gather(B, N // _PACK)
    gu, gi = gather(user.astype(jnp.int32), item.astype(jnp.int32), upk, ipk)
    W1u_big = jnp.tile(W1[:_F], (_PACK, 1))
    W1i_big = jnp.tile(W1[_F:], (_PACK, 1))
    u2d = user.astype(jnp.int32).reshape(B, 1)
    i2d = item.astype(jnp.int32).reshape(B, 1)
    return _mlp_pallas(gu, gi, u2d, i2d, W1u_big, W1i_big, b1, W2, b2, W3, b3)


# speedup vs baseline: 2.0784x; 2.0784x over previous
import functools

import jax
import jax.numpy as jnp
from jax import lax
from jax.experimental import pallas as pl
from jax.experimental.pallas import tpu as pltpu
from jax.experimental.pallas import tpu_sc as plsc

_F = 16
_PACK = 8
_IDX_CHUNK = 128


@functools.cache
def _make_gather(B, R):
    info = plsc.get_sparse_core_info()
    nc, ns = info.num_cores, info.num_subcores
    nw = nc * ns
    b_per_w = B // nw
    n_chunks = b_per_w // _IDX_CHUNK
    mesh = plsc.VectorSubcoreMesh(core_axis_name="c", subcore_axis_name="s")

    @functools.partial(
        pl.kernel,
        mesh=mesh,
        compiler_params=pltpu.CompilerParams(use_tc_tiling_on_sc=False),
        out_type=(
            jax.ShapeDtypeStruct((B, _PACK * _F), jnp.float32),
            jax.ShapeDtypeStruct((B, _PACK * _F), jnp.float32),
        ),
        scratch_types=[
            pltpu.VMEM((n_chunks, _IDX_CHUNK), jnp.int32),
            pltpu.VMEM((n_chunks, _IDX_CHUNK), jnp.int32),
            pltpu.VMEM((b_per_w, _PACK * _F), jnp.float32),
            pltpu.SemaphoreType.DMA,
        ],
    )
    def gather_k(user_hbm, item_hbm, upk_hbm, ipk_hbm, u_out, i_out,
                 idx_v, q_v, rows_v, sem):
        wid = lax.axis_index("s") * nc + lax.axis_index("c")
        base = wid * b_per_w
        for idx_hbm, pk_hbm, out in ((user_hbm, upk_hbm, u_out),
                                     (item_hbm, ipk_hbm, i_out)):
            for j in range(n_chunks):
                pltpu.sync_copy(idx_hbm.at[pl.ds(base + j * _IDX_CHUNK,
                                                 _IDX_CHUNK)], idx_v.at[j])
            for j in range(n_chunks):
                for s in range(_IDX_CHUNK // 16):
                    q_v[j, pl.ds(s * 16, 16)] = (
                        idx_v[j, pl.ds(s * 16, 16)] >> 3)
            copies = [
                pltpu.async_copy(
                    pk_hbm.at[q_v.at[j]],
                    rows_v.at[pl.ds(j * _IDX_CHUNK, _IDX_CHUNK)], sem)
                for j in range(n_chunks)
            ]
            for c in copies:
                c.wait()
            pltpu.sync_copy(rows_v, out.at[pl.ds(base, b_per_w)])

    return gather_k


def _mlp_pallas(gu, gi, u2d, i2d, W1u_big, W1i_big, b1, W2, b2, W3, b3):
    B, L = gu.shape
    blk = 2048
    n1 = W1u_big.shape[1]
    n2 = W2.shape[1]

    def body(gu_ref, gi_ref, u_ref, i_ref, w1u_ref, w1i_ref, b1_ref,
             w2_ref, b2_ref, w3_ref, b3_ref, out_ref):
        su = u_ref[...] & 7
        si = i_ref[...] & 7
        grp = lax.broadcasted_iota(jnp.int32, (blk, L), 1) >> 4
        um = jnp.where(grp == su, gu_ref[...], 0.0)
        im = jnp.where(grp == si, gi_ref[...], 0.0)
        h = (jnp.dot(um, w1u_ref[...], preferred_element_type=jnp.float32)
             + jnp.dot(im, w1i_ref[...], preferred_element_type=jnp.float32)
             + b1_ref[...])
        h = jnp.dot(h, w2_ref[...], preferred_element_type=jnp.float32) \
            + b2_ref[...]
        o = jnp.dot(h, w3_ref[...], preferred_element_type=jnp.float32) \
            + b3_ref[...]
        out_ref[...] = 1.0 / (1.0 + jnp.exp(-o))

    return pl.pallas_call(
        body,
        grid=(B // blk,),
        in_specs=[
            pl.BlockSpec((blk, L), lambda i: (i, 0)),
            pl.BlockSpec((blk, L), lambda i: (i, 0)),
            pl.BlockSpec((blk, 1), lambda i: (i, 0)),
            pl.BlockSpec((blk, 1), lambda i: (i, 0)),
            pl.BlockSpec((L, n1), lambda i: (0, 0)),
            pl.BlockSpec((L, n1), lambda i: (0, 0)),
            pl.BlockSpec((1, n1), lambda i: (0, 0)),
            pl.BlockSpec((n1, n2), lambda i: (0, 0)),
            pl.BlockSpec((1, n2), lambda i: (0, 0)),
            pl.BlockSpec((n2, 1), lambda i: (0, 0)),
            pl.BlockSpec((1, 1), lambda i: (0, 0)),
        ],
        out_specs=pl.BlockSpec((blk, 1), lambda i: (i, 0)),
        out_shape=jax.ShapeDtypeStruct((B, 1), jnp.float32),
    )(gu, gi, u2d, i2d, W1u_big, W1i_big, b1.reshape(1, n1), W2,
      b2.reshape(1, n2), W3, b3.reshape(1, 1))


def kernel(user, item, user_emb, item_emb, W1, b1, W2, b2, W3, b3):
    B = user.shape[0]
    N = user_emb.shape[0]
    upk = user_emb.reshape(N // _PACK, _PACK * _F)
    ipk = item_emb.reshape(N // _PACK, _PACK * _F)
    gather = _make_gather(B, N // _PACK)
    gu, gi = gather(user.astype(jnp.int32), item.astype(jnp.int32), upk, ipk)
    W1u_big = jnp.tile(W1[:_F], (_PACK, 1))
    W1i_big = jnp.tile(W1[_F:], (_PACK, 1))
    u2d = user.astype(jnp.int32).reshape(B, 1)
    i2d = item.astype(jnp.int32).reshape(B, 1)
    return _mlp_pallas(gu, gi, u2d, i2d, W1u_big, W1i_big, b1, W2, b2, W3, b3)
